# baseline (speedup 1.0000x reference)
import functools

import jax
import jax.numpy as jnp
from jax import lax
from jax.experimental import pallas as pl
from jax.experimental.pallas import tpu as pltpu
from jax.experimental.pallas import tpu_sc as plsc

_NC = 2
_NS = 16
_NW = _NC * _NS
_LANES = 16


def _positional_encoding(static_len: int, dims: int) -> jnp.ndarray:
    pos = jnp.arange(static_len, dtype=jnp.float32)[:, None]
    i = jnp.arange(dims, dtype=jnp.float32)[None, :]
    angle = pos / jnp.power(10000.0, 2.0 * i / dims)
    even = jnp.sin(angle)
    odd = jnp.cos(angle)
    col = jnp.arange(dims)[None, :]
    pe = jnp.where(col % 2 == 0, even, odd)
    pe = pe.at[0].set(0.0)
    return pe


@functools.partial(jax.jit, static_argnames=("n_rows", "dims", "chunk", "n_chunks"))
def _sc_embed(table, idx3, pe_tile, *, n_rows, dims, chunk, n_chunks):
    rows_per_w = n_rows // _NW
    mesh = plsc.VectorSubcoreMesh(
        core_axis_name="c", subcore_axis_name="s", num_cores=_NC, num_subcores=_NS
    )

    @functools.partial(
        pl.kernel,
        out_type=jax.ShapeDtypeStruct((n_rows, dims), jnp.float32),
        mesh=mesh,
        scratch_types=[
            pltpu.VMEM((n_chunks * chunk,), jnp.int32),
            pltpu.VMEM((chunk, dims), jnp.float32),
            pltpu.VMEM((chunk, dims), jnp.float32),
            pltpu.SemaphoreType.DMA,
        ],
        compiler_params=pltpu.CompilerParams(use_tc_tiling_on_sc=False),
    )
    def body(table_hbm, idx_hbm, pe_hbm, out_hbm, idx_v, pe_v, rows_v, sem):
        wid = lax.axis_index("s") * _NC + lax.axis_index("c")
        base = wid * rows_per_w
        pltpu.sync_copy(idx_hbm.at[wid], idx_v)
        pltpu.sync_copy(pe_hbm, pe_v)

        @pl.loop(0, n_chunks)
        def _chunk_loop(c):
            pltpu.async_copy(
                table_hbm.at[idx_v.at[pl.ds(c * chunk, chunk)]], rows_v, sem
            ).wait()

            @pl.loop(0, chunk)
            def _row_loop(r):
                for j in range(dims // _LANES):
                    sl = pl.ds(j * _LANES, _LANES)
                    rows_v[r, sl] = rows_v[r, sl] + pe_v[r, sl]

            pltpu.sync_copy(rows_v, out_hbm.at[pl.ds(base + c * chunk, chunk)])

    return body(table, idx3, pe_tile)


def kernel(x, cutoff_max_sen_len, vocab_size, table):
    batch, seq = x.shape
    _, dims = table.shape
    n_rows = batch * seq

    chunk = 400
    assert chunk % seq == 0 and n_rows % (_NW * chunk) == 0
    n_chunks = n_rows // (_NW * chunk)

    pe = _positional_encoding(seq, dims)
    pe_tile = jnp.tile(pe, (chunk // seq, 1))
    idx3 = x.reshape(_NW, n_chunks * chunk)

    out = _sc_embed(
        table, idx3, pe_tile, n_rows=n_rows, dims=dims, chunk=chunk, n_chunks=n_chunks
    )
    return out.reshape(batch, seq, dims)
